# Initial kernel scaffold; baseline (speedup 1.0000x reference)
import jax
import jax.numpy as jnp
from jax.experimental import pallas as pl


def kernel(preds, labels):
    raise NotImplementedError("write your pallas kernel here")


# speedup vs baseline: 1.3537x; 1.3537x over previous
import jax
import jax.numpy as jnp
from jax.experimental import pallas as pl
from jax.experimental.pallas import tpu as pltpu

_B = 8


def _body(p_ref, l_ref, out_ref):
    p = p_ref[...]
    l = l_ref[...]
    g = (p[:, :, None] > p[:, None, :]).astype(jnp.float32)
    n = jnp.sum(g, axis=1)
    ppos = jnp.sum(g * l[:, :, None], axis=1)
    term = l * (1.0 + ppos) / (1.0 + n)
    num = jnp.sum(term)
    den = jnp.sum(l)
    lane = jax.lax.broadcasted_iota(jnp.int32, (1, 128), 1)
    contrib = jnp.where(lane == 0, num, 0.0) + jnp.where(lane == 1, den, 0.0)

    @pl.when(pl.program_id(0) == 0)
    def _init():
        out_ref[...] = jnp.zeros_like(out_ref)

    out_ref[...] += contrib


def kernel(preds, labels):
    R, C = preds.shape
    grid = R // _B
    out = pl.pallas_call(
        _body,
        grid=(grid,),
        in_specs=[
            pl.BlockSpec((_B, C), lambda i: (i, 0)),
            pl.BlockSpec((_B, C), lambda i: (i, 0)),
        ],
        out_specs=pl.BlockSpec((1, 128), lambda i: (0, 0)),
        out_shape=jax.ShapeDtypeStruct((1, 128), jnp.float32),
    )(preds, labels)
    return out[0, 0] / out[0, 1]


# speedup vs baseline: 12.7703x; 9.4334x over previous
import functools

import jax
import jax.numpy as jnp
from jax import lax
from jax.experimental import pallas as pl
from jax.experimental.pallas import tpu as pltpu
from jax.experimental.pallas import tpu_sc as plsc

_NC, _NS, _L = 2, 16, 16
_NW = _NC * _NS
_R, _C = 4096, 512
_V = _C // _L
_RPW = _R // _NW
_RB = 64
_NBLK = _RPW // _RB


def _row_score(ks, wbuf, num_vec, den_vec):
    for i in range(_V):
        d = (i % 2 == 1)
        ks[i] = plsc.sort_key_val(ks[i], ks[i], descending=d)[0]
    for K in (2, 4, 8, 16, 32):
        J = K // 2
        while J >= 1:
            for b in range(_V):
                if b & J == 0:
                    q = b | J
                    asc = (b & K) == 0
                    ka, kb = ks[b], ks[q]
                    swap = (ka > kb) if asc else (ka < kb)
                    ks[b] = jnp.where(swap, kb, ka)
                    ks[q] = jnp.where(swap, ka, kb)
            J //= 2
        for i in range(_V):
            d = (i & K) != 0
            ks[i] = plsc.sort_key_val(ks[i], ks[i], descending=d)[0]
    carry = jnp.zeros((_L,), jnp.float32)
    for i in range(_V):
        sl = (ks[i] & 1).astype(jnp.float32)
        pre = plsc.cumsum(sl)
        w = wbuf[pl.ds(i * _L, _L)]
        num_vec = num_vec + sl * (carry + pre) * w
        den_vec = den_vec + sl
        if i < _V - 1:
            carry = carry + jnp.sum(sl)
    return num_vec, den_vec


def _sc_body(preds_hbm, labels_hbm, out_hbm, pbuf, lbuf, wbuf, obuf):
    wid = lax.axis_index("s") * _NC + lax.axis_index("c")
    iota_f = lax.iota(jnp.int32, _L).astype(jnp.float32)
    for i in range(_V):
        wbuf[pl.ds(i * _L, _L)] = 1.0 / (iota_f + float(i * _L + 1))
    row0 = wid * _RPW

    def blk_body(blk, carry):
        num_vec, den_vec = carry
        off = (row0 + blk * _RB) * _C
        pltpu.sync_copy(preds_hbm.at[pl.ds(off, _RB * _C)], pbuf)
        pltpu.sync_copy(labels_hbm.at[pl.ds(off, _RB * _C)], lbuf)

        def row_body(r, carry2):
            base = r * _C
            ks = []
            for i in range(_V):
                p = pbuf[pl.ds(base + i * _L, _L)]
                l = lbuf[pl.ds(base + i * _L, _L)]
                s = lax.bitcast_convert_type(p, jnp.int32)
                m = s ^ ((s >> 31) & jnp.int32(0x7FFFFFFF))
                li = l.astype(jnp.int32)
                ks.append(((-m) & jnp.int32(-2)) | li)
            return _row_score(ks, wbuf, *carry2)

        return lax.fori_loop(0, _RB, row_body, (num_vec, den_vec))

    zeros = jnp.zeros((_L,), jnp.float32)
    num_vec, den_vec = lax.fori_loop(0, _NBLK, blk_body, (zeros, zeros))
    obuf[pl.ds(0, _L)] = num_vec
    obuf[pl.ds(_L, _L)] = den_vec
    pltpu.sync_copy(obuf, out_hbm.at[pl.ds(wid * 2 * _L, 2 * _L)])


def kernel(preds, labels):
    mesh = plsc.VectorSubcoreMesh(
        core_axis_name="c", subcore_axis_name="s",
        num_cores=_NC, num_subcores=_NS)
    k = functools.partial(
        pl.kernel,
        out_type=jax.ShapeDtypeStruct((_NW * 2 * _L,), jnp.float32),
        mesh=mesh,
        compiler_params=pltpu.CompilerParams(needs_layout_passes=False),
        scratch_types=[
            pltpu.VMEM((_RB * _C,), jnp.float32),
            pltpu.VMEM((_RB * _C,), jnp.float32),
            pltpu.VMEM((_C,), jnp.float32),
            pltpu.VMEM((2 * _L,), jnp.float32),
        ],
    )(_sc_body)
    out = k(preds.reshape(-1), labels.reshape(-1))
    o = out.reshape(_NW, 2, _L)
    return o[:, 0].sum() / o[:, 1].sum()


# speedup vs baseline: 14.1579x; 1.1087x over previous
import functools

import jax
import jax.numpy as jnp
from jax import lax
from jax.experimental import pallas as pl
from jax.experimental.pallas import tpu as pltpu
from jax.experimental.pallas import tpu_sc as plsc

_NC, _NS, _L = 2, 16, 16
_NW = _NC * _NS
_R, _C = 4096, 512
_V = _C // _L
_RPW = _R // _NW
_RB = 64
_NBLK = _RPW // _RB


def _row_score(ks, wbuf, num_vec, den_vec):
    for i in range(_V):
        d = (i % 2 == 1)
        ks[i] = plsc.sort_key_val(ks[i], ks[i], descending=d)[0]
    for K in (2, 4, 8, 16, 32):
        J = K // 2
        while J >= 1:
            for b in range(_V):
                if b & J == 0:
                    q = b | J
                    ka, kb = ks[b], ks[q]
                    lo = jnp.minimum(ka, kb)
                    hi = jnp.maximum(ka, kb)
                    if (b & K) == 0:
                        ks[b], ks[q] = lo, hi
                    else:
                        ks[b], ks[q] = hi, lo
            J //= 2
        for i in range(_V):
            d = (i & K) != 0
            ks[i] = plsc.sort_key_val(ks[i], ks[i], descending=d)[0]
    carry = jnp.zeros((_L,), jnp.float32)
    one = jnp.uint32(1)
    for i in range(_V):
        lbit = ks[i] & one
        sl = lbit.astype(jnp.float32)
        pre = plsc.cumsum(sl)
        w = wbuf[pl.ds(i * _L, _L)]
        num_vec = num_vec + sl * (carry + pre) * w
        den_vec = den_vec + sl
        if i < _V - 1:
            cnt = plsc.all_reduce_population_count(lbit != 0)
            carry = carry + cnt.astype(jnp.float32)
    return num_vec, den_vec


def _sc_body(preds_hbm, labels_hbm, out_hbm, pbuf, lbuf, wbuf, obuf):
    wid = lax.axis_index("s") * _NC + lax.axis_index("c")
    iota_f = lax.iota(jnp.int32, _L).astype(jnp.float32)
    for i in range(_V):
        wbuf[pl.ds(i * _L, _L)] = 1.0 / (iota_f + float(i * _L + 1))
    row0 = wid * _RPW

    def blk_body(blk, carry):
        num_vec, den_vec = carry
        off = (row0 + blk * _RB) * _C
        pltpu.sync_copy(preds_hbm.at[pl.ds(off, _RB * _C)], pbuf)
        pltpu.sync_copy(labels_hbm.at[pl.ds(off, _RB * _C)], lbuf)

        def row_body(r, carry2):
            base = r * _C
            ks = []
            for i in range(_V):
                p = pbuf[pl.ds(base + i * _L, _L)]
                l = lbuf[pl.ds(base + i * _L, _L)]
                s = lax.bitcast_convert_type(p, jnp.int32)
                ud = s ^ ((~s >> 31) & jnp.int32(0x7FFFFFFF))
                li = l.astype(jnp.int32)
                k = (ud & jnp.int32(-2)) | li
                ks.append(lax.bitcast_convert_type(k, jnp.uint32))
            return _row_score(ks, wbuf, *carry2)

        return lax.fori_loop(0, _RB, row_body, (num_vec, den_vec))

    zeros = jnp.zeros((_L,), jnp.float32)
    num_vec, den_vec = lax.fori_loop(0, _NBLK, blk_body, (zeros, zeros))
    obuf[pl.ds(0, _L)] = num_vec
    obuf[pl.ds(_L, _L)] = den_vec
    pltpu.sync_copy(obuf, out_hbm.at[pl.ds(wid * 2 * _L, 2 * _L)])


def kernel(preds, labels):
    mesh = plsc.VectorSubcoreMesh(
        core_axis_name="c", subcore_axis_name="s",
        num_cores=_NC, num_subcores=_NS)
    k = functools.partial(
        pl.kernel,
        out_type=jax.ShapeDtypeStruct((_NW * 2 * _L,), jnp.float32),
        mesh=mesh,
        compiler_params=pltpu.CompilerParams(needs_layout_passes=False),
        scratch_types=[
            pltpu.VMEM((_RB * _C,), jnp.float32),
            pltpu.VMEM((_RB * _C,), jnp.float32),
            pltpu.VMEM((_C,), jnp.float32),
            pltpu.VMEM((2 * _L,), jnp.float32),
        ],
    )(_sc_body)
    out = k(preds.reshape(-1), labels.reshape(-1))
    o = out.reshape(_NW, 2, _L)
    return o[:, 0].sum() / o[:, 1].sum()


# speedup vs baseline: 18.7822x; 1.3266x over previous
import functools

import jax
import jax.numpy as jnp
from jax import lax
from jax.experimental import pallas as pl
from jax.experimental.pallas import tpu as pltpu
from jax.experimental.pallas import tpu_sc as plsc

_NC, _NS, _L = 2, 16, 16
_NW = _NC * _NS
_R, _C = 4096, 512
_V = _C // _L
_RPW = _R // _NW
_RB = 64
_NBLK = _RPW // _RB


def _row_score(ks, wbuf, num_vec, den_vec):
    for i in range(_V):
        d = (i % 2 == 1)
        ks[i] = plsc.sort_key_val(ks[i], ks[i], descending=d)[0]
    for K in (2, 4, 8, 16, 32):
        J = K // 2
        while J >= 1:
            for b in range(_V):
                if b & J == 0:
                    q = b | J
                    ka, kb = ks[b], ks[q]
                    lo = jnp.minimum(ka, kb)
                    hi = jnp.maximum(ka, kb)
                    if (b & K) == 0:
                        ks[b], ks[q] = lo, hi
                    else:
                        ks[b], ks[q] = hi, lo
            J //= 2
        for i in range(_V):
            d = (i & K) != 0
            ks[i] = plsc.sort_key_val(ks[i], ks[i], descending=d)[0]
    carry = jnp.zeros((_L,), jnp.float32)
    one = jnp.uint32(1)
    for i in range(_V):
        lbit = ks[i] & one
        sl = lbit.astype(jnp.float32)
        pre = plsc.cumsum(sl)
        w = wbuf[pl.ds(i * _L, _L)]
        num_vec = num_vec + sl * (carry + pre) * w
        den_vec = den_vec + sl
        if i < _V - 1:
            cnt = plsc.all_reduce_population_count(lbit != 0)
            carry = carry + cnt.astype(jnp.float32)
    return num_vec, den_vec


def _sc_body(preds_hbm, labels_hbm, out_hbm, pbuf, lbuf, wbuf, obuf):
    wid = lax.axis_index("s") * _NC + lax.axis_index("c")
    iota_f = lax.iota(jnp.int32, _L).astype(jnp.float32)
    for i in range(_V):
        wbuf[pl.ds(i * _L, _L)] = 1.0 / (iota_f + float(i * _L + 1))
    row0 = wid * _RPW

    def blk_body(blk, carry):
        num_vec, den_vec = carry
        off = row0 + blk * _RB
        pltpu.sync_copy(preds_hbm.at[pl.ds(off, _RB)], pbuf)
        pltpu.sync_copy(labels_hbm.at[pl.ds(off, _RB)], lbuf)

        def row_body(r, carry2):
            ks = []
            for i in range(_V):
                p = pbuf[r, pl.ds(i * _L, _L)]
                l = lbuf[r, pl.ds(i * _L, _L)]
                s = lax.bitcast_convert_type(p, jnp.int32)
                ud = s ^ ((~s >> 31) & jnp.int32(0x7FFFFFFF))
                li = l.astype(jnp.int32)
                k = (ud & jnp.int32(-2)) | li
                ks.append(lax.bitcast_convert_type(k, jnp.uint32))
            return _row_score(ks, wbuf, *carry2)

        return lax.fori_loop(0, _RB, row_body, (num_vec, den_vec))

    zeros = jnp.zeros((_L,), jnp.float32)
    num_vec, den_vec = lax.fori_loop(0, _NBLK, blk_body, (zeros, zeros))
    obuf[pl.ds(0, _L)] = num_vec
    obuf[pl.ds(_L, _L)] = den_vec
    pltpu.sync_copy(obuf, out_hbm.at[pl.ds(wid * 2 * _L, 2 * _L)])


def kernel(preds, labels):
    mesh = plsc.VectorSubcoreMesh(
        core_axis_name="c", subcore_axis_name="s",
        num_cores=_NC, num_subcores=_NS)
    k = functools.partial(
        pl.kernel,
        out_type=jax.ShapeDtypeStruct((_NW * 2 * _L,), jnp.float32),
        mesh=mesh,
        compiler_params=pltpu.CompilerParams(needs_layout_passes=False),
        scratch_types=[
            pltpu.VMEM((_RB, _C), jnp.float32),
            pltpu.VMEM((_RB, _C), jnp.float32),
            pltpu.VMEM((_C,), jnp.float32),
            pltpu.VMEM((2 * _L,), jnp.float32),
        ],
    )(_sc_body)
    out = k(preds, labels)
    o = out.reshape(_NW, 2, _L)
    return o[:, 0].sum() / o[:, 1].sum()


# speedup vs baseline: 19.2910x; 1.0271x over previous
import functools

import jax
import jax.numpy as jnp
from jax import lax
from jax.experimental import pallas as pl
from jax.experimental.pallas import tpu as pltpu
from jax.experimental.pallas import tpu_sc as plsc

_NC, _NS, _L = 2, 16, 16
_NW = _NC * _NS
_R, _C = 4096, 512
_V = _C // _L
_RPW = _R // _NW
_RB = 32
_NBLK = _RPW // _RB


def _row_score(ks, wbuf, num_vec, den_vec):
    for i in range(_V):
        d = (i % 2 == 1)
        ks[i] = plsc.sort_key_val(ks[i], ks[i], descending=d)[0]
    for K in (2, 4, 8, 16, 32):
        J = K // 2
        while J >= 1:
            for b in range(_V):
                if b & J == 0:
                    q = b | J
                    ka, kb = ks[b], ks[q]
                    lo = jnp.minimum(ka, kb)
                    hi = jnp.maximum(ka, kb)
                    if (b & K) == 0:
                        ks[b], ks[q] = lo, hi
                    else:
                        ks[b], ks[q] = hi, lo
            J //= 2
        for i in range(_V):
            d = (i & K) != 0
            ks[i] = plsc.sort_key_val(ks[i], ks[i], descending=d)[0]
    carry = jnp.zeros((_L,), jnp.float32)
    one = jnp.uint32(1)
    for i in range(_V):
        lbit = ks[i] & one
        sl = lbit.astype(jnp.float32)
        pre = plsc.cumsum(sl)
        w = wbuf[pl.ds(i * _L, _L)]
        num_vec = num_vec + sl * (carry + pre) * w
        den_vec = den_vec + sl
        if i < _V - 1:
            cnt = plsc.all_reduce_population_count(lbit != 0)
            carry = carry + cnt.astype(jnp.float32)
    return num_vec, den_vec


def _sc_body(preds_hbm, labels_hbm, out_hbm,
             pbufs, lbufs, wbuf, obuf, sems):
    wid = lax.axis_index("s") * _NC + lax.axis_index("c")
    iota_f = lax.iota(jnp.int32, _L).astype(jnp.float32)
    for i in range(_V):
        wbuf[pl.ds(i * _L, _L)] = 1.0 / (iota_f + float(i * _L + 1))
    row0 = wid * _RPW

    def copies(blk, slot):
        off = row0 + blk * _RB
        return (
            pltpu.make_async_copy(
                preds_hbm.at[pl.ds(off, _RB)], pbufs[slot], sems[slot]),
            pltpu.make_async_copy(
                labels_hbm.at[pl.ds(off, _RB)], lbufs[slot], sems[slot]),
        )

    def row_body_for(pbuf, lbuf):
        def row_body(r, carry2):
            ks = []
            for i in range(_V):
                p = pbuf[r, pl.ds(i * _L, _L)]
                l = lbuf[r, pl.ds(i * _L, _L)]
                s = lax.bitcast_convert_type(p, jnp.int32)
                ud = s ^ ((~s >> 31) & jnp.int32(0x7FFFFFFF))
                li = l.astype(jnp.int32)
                k = (ud & jnp.int32(-2)) | li
                ks.append(lax.bitcast_convert_type(k, jnp.uint32))
            return _row_score(ks, wbuf, *carry2)
        return row_body

    for c in copies(0, 0):
        c.start()
    zeros = jnp.zeros((_L,), jnp.float32)
    carry = (zeros, zeros)
    for blk in range(_NBLK):
        slot = blk % 2
        if blk + 1 < _NBLK:
            nxt = copies(blk + 1, 1 - slot)
            for c in nxt:
                c.start()
        for c in copies(blk, slot):
            c.wait()
        carry = lax.fori_loop(
            0, _RB, row_body_for(pbufs[slot], lbufs[slot]), carry)
    num_vec, den_vec = carry
    obuf[pl.ds(0, _L)] = num_vec
    obuf[pl.ds(_L, _L)] = den_vec
    pltpu.sync_copy(obuf, out_hbm.at[pl.ds(wid * 2 * _L, 2 * _L)])


def kernel(preds, labels):
    mesh = plsc.VectorSubcoreMesh(
        core_axis_name="c", subcore_axis_name="s",
        num_cores=_NC, num_subcores=_NS)
    k = functools.partial(
        pl.kernel,
        out_type=jax.ShapeDtypeStruct((_NW * 2 * _L,), jnp.float32),
        mesh=mesh,
        compiler_params=pltpu.CompilerParams(needs_layout_passes=False),
        scratch_types=[
            [pltpu.VMEM((_RB, _C), jnp.float32)] * 2,
            [pltpu.VMEM((_RB, _C), jnp.float32)] * 2,
            pltpu.VMEM((_C,), jnp.float32),
            pltpu.VMEM((2 * _L,), jnp.float32),
            [pltpu.SemaphoreType.DMA] * 2,
        ],
    )(_sc_body)
    out = k(preds, labels)
    o = out.reshape(_NW, 2, _L)
    return o[:, 0].sum() / o[:, 1].sum()


# speedup vs baseline: 19.9987x; 1.0367x over previous
import functools

import jax
import jax.numpy as jnp
from jax import lax
from jax.experimental import pallas as pl
from jax.experimental.pallas import tpu as pltpu
from jax.experimental.pallas import tpu_sc as plsc

_NC, _NS, _L = 2, 16, 16
_NW = _NC * _NS
_R, _C = 4096, 512
_V = _C // _L
_RPW = _R // _NW
_RB = 32
_NBLK = _RPW // _RB


def _row_score(ks, wbuf, num_vec, den_vec):
    for i in range(_V):
        d = (i % 2 == 0)
        ks[i] = plsc.sort_key_val(ks[i], ks[i], descending=d)[0]
    for K in (2, 4, 8, 16, 32):
        J = K // 2
        while J >= 1:
            for b in range(_V):
                if b & J == 0:
                    q = b | J
                    ka, kb = ks[b], ks[q]
                    lo = jnp.minimum(ka, kb)
                    hi = jnp.maximum(ka, kb)
                    if (b & K) == 0:
                        ks[b], ks[q] = hi, lo
                    else:
                        ks[b], ks[q] = lo, hi
            J //= 2
        for i in range(_V):
            d = (i & K) == 0
            ks[i] = plsc.sort_key_val(ks[i], ks[i], descending=d)[0]
    carry = jnp.zeros((_L,), jnp.float32)
    one = jnp.int32(1)
    for i in range(_V):
        lbit = lax.bitcast_convert_type(ks[i], jnp.int32) & one
        sl = lbit.astype(jnp.float32)
        pre = plsc.cumsum(sl)
        w = wbuf[pl.ds(i * _L, _L)]
        num_vec = num_vec + sl * (carry + pre) * w
        cnt = plsc.all_reduce_population_count(lbit != 0)
        carry = carry + cnt.astype(jnp.float32)
    return num_vec, den_vec + carry


def _sc_body(preds_hbm, labels_hbm, out_hbm,
             pbufs, lbufs, wbuf, obuf, sems):
    wid = lax.axis_index("s") * _NC + lax.axis_index("c")
    iota_f = lax.iota(jnp.int32, _L).astype(jnp.float32)
    for i in range(_V):
        wbuf[pl.ds(i * _L, _L)] = 1.0 / (iota_f + float(i * _L + 1))
    row0 = wid * _RPW

    def copies(blk, slot):
        off = row0 + blk * _RB
        return (
            pltpu.make_async_copy(
                preds_hbm.at[pl.ds(off, _RB)], pbufs[slot], sems[slot]),
            pltpu.make_async_copy(
                labels_hbm.at[pl.ds(off, _RB)], lbufs[slot], sems[slot]),
        )

    def row_body_for(pbuf, lbuf):
        def row_body(r, carry2):
            ks = []
            for i in range(_V):
                p = pbuf[r, pl.ds(i * _L, _L)]
                l = lbuf[r, pl.ds(i * _L, _L)]
                s = lax.bitcast_convert_type(p, jnp.int32)
                li = l.astype(jnp.int32)
                k = (s & jnp.int32(-2)) | li
                ks.append(lax.bitcast_convert_type(k, jnp.float32))
            return _row_score(ks, wbuf, *carry2)
        return row_body

    for c in copies(0, 0):
        c.start()
    zeros = jnp.zeros((_L,), jnp.float32)
    carry = (zeros, zeros)
    for blk in range(_NBLK):
        slot = blk % 2
        if blk + 1 < _NBLK:
            nxt = copies(blk + 1, 1 - slot)
            for c in nxt:
                c.start()
        for c in copies(blk, slot):
            c.wait()
        carry = lax.fori_loop(
            0, _RB, row_body_for(pbufs[slot], lbufs[slot]), carry)
    num_vec, den_vec = carry
    obuf[pl.ds(0, _L)] = num_vec
    obuf[pl.ds(_L, _L)] = den_vec * (1.0 / _L)
    pltpu.sync_copy(obuf, out_hbm.at[pl.ds(wid * 2 * _L, 2 * _L)])


def kernel(preds, labels):
    mesh = plsc.VectorSubcoreMesh(
        core_axis_name="c", subcore_axis_name="s",
        num_cores=_NC, num_subcores=_NS)
    k = functools.partial(
        pl.kernel,
        out_type=jax.ShapeDtypeStruct((_NW * 2 * _L,), jnp.float32),
        mesh=mesh,
        compiler_params=pltpu.CompilerParams(needs_layout_passes=False),
        scratch_types=[
            [pltpu.VMEM((_RB, _C), jnp.float32)] * 2,
            [pltpu.VMEM((_RB, _C), jnp.float32)] * 2,
            pltpu.VMEM((_C,), jnp.float32),
            pltpu.VMEM((2 * _L,), jnp.float32),
            [pltpu.SemaphoreType.DMA] * 2,
        ],
    )(_sc_body)
    out = k(preds, labels)
    o = out.reshape(_NW, 2, _L)
    return o[:, 0].sum() / o[:, 1].sum()
